# baseline (speedup 1.0000x reference)
import functools

import jax
import jax.numpy as jnp
from jax import lax
from jax.experimental import pallas as pl
from jax.experimental.pallas import tpu as pltpu
from jax.experimental.pallas import tpu_sc as plsc

FEAT = 32
VOCAB = 100000
CH = 4096
K = 8


def kernel(element_list, encodings):
    idx = element_list.reshape(-1).astype(jnp.int32)
    B = idx.shape[0]
    table_t = encodings.T

    info = plsc.get_sparse_core_info()
    nc, ns = info.num_cores, info.num_subcores
    nch = B // CH
    nslab = nch // K

    mesh = plsc.VectorSubcoreMesh(core_axis_name="c", subcore_axis_name="s")

    @functools.partial(
        pl.kernel,
        mesh=mesh,
        out_type=jax.ShapeDtypeStruct((FEAT, B), jnp.float32),
        scratch_types=[
            pltpu.VMEM_SHARED((2, K * CH), jnp.int32),
            pltpu.VMEM((VOCAB,), jnp.float32),
            pltpu.VMEM((CH,), jnp.int32),
            pltpu.VMEM((CH,), jnp.int32),
            pltpu.VMEM((CH,), jnp.float32),
            pltpu.VMEM((CH,), jnp.float32),
            pltpu.SemaphoreType.DMA,
            pltpu.SemaphoreType.DMA,
            pltpu.SemaphoreType.DMA,
            pltpu.SemaphoreType.DMA,
            pltpu.SemaphoreType.DMA,
        ],
        compiler_params=pltpu.CompilerParams(needs_layout_passes=False),
    )
    def gather_kernel(idx_hbm, table_hbm, out_hbm, sh_idx, tab_v, idx_v0,
                      idx_v1, out_v0, out_v1, sem_i0, sem_i1, sem_o0,
                      sem_o1, sem_slab):
        c = lax.axis_index("c")
        s = lax.axis_index("s")
        f = c * ns + s
        sem_i = (sem_i0, sem_i1)
        sem_o = (sem_o0, sem_o1)
        idx_v = (idx_v0, idx_v1)
        out_v = (out_v0, out_v1)

        def slab_fetch_start(si, a):
            pltpu.async_copy(idx_hbm.at[pl.ds(si * (K * CH), K * CH)],
                             sh_idx.at[a], sem_slab)

        def slab_fetch_wait(a):
            pltpu.make_async_copy(idx_hbm.at[pl.ds(0, K * CH)],
                                  sh_idx.at[a], sem_slab).wait()

        def start_idx(kk, b, a):
            pltpu.async_copy(sh_idx.at[a, pl.ds(kk * CH, CH)], idx_v[b],
                             sem_i[b])

        def wait_idx(b):
            pltpu.make_async_copy(idx_hbm.at[pl.ds(0, CH)], idx_v[b],
                                  sem_i[b]).wait()

        def compute(b):
            ib = idx_v[b]
            ob = out_v[b]

            @plsc.parallel_loop(0, CH, step=16, unroll=32)
            def _(j):
                iv = ib[pl.ds(j, 16)]
                ob[pl.ds(j, 16)] = plsc.load_gather(tab_v, [iv])

        def start_out(m, b):
            pltpu.async_copy(out_v[b],
                             out_hbm.at[f, pl.ds(m * CH, CH)],
                             sem_o[b])

        def wait_out(b):
            pltpu.make_async_copy(out_v[b],
                                  out_hbm.at[f, pl.ds(0, CH)],
                                  sem_o[b]).wait()

        def chunk_in_slab(k):
            return lax.rem(k + s, K)

        def do_slab(si, a, prefetch, first, last):
            if prefetch:
                @pl.when(s == 0)
                def _():
                    slab_fetch_start(si + 1, 1 - a)

            kk0 = chunk_in_slab(0)
            start_idx(kk0, 0, a)
            for k in range(K):
                b = k % 2
                wait_idx(b)
                if k + 1 < K:
                    start_idx(chunk_in_slab(k + 1), 1 - b, a)
                if not first or k >= 2:
                    wait_out(b)
                compute(b)
                start_out(si * K + chunk_in_slab(k), b)

            if prefetch:
                @pl.when(s == 0)
                def _():
                    slab_fetch_wait(1 - a)

            if not last:
                plsc.subcore_barrier()

        @pl.when(s == 0)
        def _():
            slab_fetch_start(0, 0)

        pltpu.sync_copy(table_hbm.at[f], tab_v)

        @pl.when(s == 0)
        def _():
            slab_fetch_wait(0)

        plsc.subcore_barrier()

        do_slab(jnp.int32(0), 0, True, True, False)

        @pl.loop(1, nslab - 2, step=2)
        def _(t):
            do_slab(t, 1, True, False, False)
            do_slab(t + 1, 0, True, False, False)

        do_slab(jnp.int32(nslab - 2), 1, True, False, False)
        do_slab(jnp.int32(nslab - 1), 0, False, False, True)

        for b in range(2):
            wait_out(b)

    out_t = gather_kernel(idx, table_t)
    return out_t.T


# speedup vs baseline: 1.0277x; 1.0277x over previous
import functools

import jax
import jax.numpy as jnp
from jax import lax
from jax.experimental import pallas as pl
from jax.experimental.pallas import tpu as pltpu
from jax.experimental.pallas import tpu_sc as plsc

FEAT = 32
VOCAB = 100000
CH = 4096
K = 8


def kernel(element_list, encodings):
    idx = element_list.reshape(-1).astype(jnp.int32)
    B = idx.shape[0]
    table_t = encodings.T

    info = plsc.get_sparse_core_info()
    nc, ns = info.num_cores, info.num_subcores
    nch = B // CH
    nslab = nch // K

    mesh = plsc.VectorSubcoreMesh(core_axis_name="c", subcore_axis_name="s")

    @functools.partial(
        pl.kernel,
        mesh=mesh,
        out_type=jax.ShapeDtypeStruct((FEAT, B), jnp.float32),
        scratch_types=[
            pltpu.VMEM_SHARED((2, K * CH), jnp.int32),
            pltpu.VMEM((VOCAB,), jnp.float32),
            pltpu.VMEM((CH,), jnp.int32),
            pltpu.VMEM((CH,), jnp.int32),
            pltpu.VMEM((CH,), jnp.float32),
            pltpu.VMEM((CH,), jnp.float32),
            pltpu.SemaphoreType.DMA,
            pltpu.SemaphoreType.DMA,
            pltpu.SemaphoreType.DMA,
            pltpu.SemaphoreType.DMA,
            pltpu.SemaphoreType.DMA,
        ],
        compiler_params=pltpu.CompilerParams(needs_layout_passes=False),
    )
    def gather_kernel(idx_hbm, table_hbm, out_hbm, sh_idx, tab_v, idx_v0,
                      idx_v1, out_v0, out_v1, sem_i0, sem_i1, sem_o0,
                      sem_o1, sem_slab):
        c = lax.axis_index("c")
        s = lax.axis_index("s")
        f = c * ns + s
        sem_i = (sem_i0, sem_i1)
        sem_o = (sem_o0, sem_o1)
        idx_v = (idx_v0, idx_v1)
        out_v = (out_v0, out_v1)

        def slab_fetch_start(si, a):
            pltpu.async_copy(idx_hbm.at[pl.ds(si * (K * CH), K * CH)],
                             sh_idx.at[a], sem_slab)

        def slab_fetch_wait(a):
            pltpu.make_async_copy(idx_hbm.at[pl.ds(0, K * CH)],
                                  sh_idx.at[a], sem_slab).wait()

        def start_idx(kk, b, a):
            pltpu.async_copy(sh_idx.at[a, pl.ds(kk * CH, CH)], idx_v[b],
                             sem_i[b])

        def wait_idx(b):
            pltpu.make_async_copy(idx_hbm.at[pl.ds(0, CH)], idx_v[b],
                                  sem_i[b]).wait()

        def compute(b):
            ib = idx_v[b]
            ob = out_v[b]

            @plsc.parallel_loop(0, CH, step=16, unroll=16)
            def _(j):
                iv = ib[pl.ds(j, 16)]
                ob[pl.ds(j, 16)] = plsc.load_gather(tab_v, [iv])

        def start_out(m, b):
            pltpu.async_copy(out_v[b],
                             out_hbm.at[f, pl.ds(m * CH, CH)],
                             sem_o[b])

        def wait_out(b):
            pltpu.make_async_copy(out_v[b],
                                  out_hbm.at[f, pl.ds(0, CH)],
                                  sem_o[b]).wait()

        def chunk_in_slab(k):
            return lax.rem(k + s, K)

        def do_slab(si, a, prefetch, first, last):
            if prefetch:
                @pl.when(s == 0)
                def _():
                    slab_fetch_start(si + 1, 1 - a)

            kk0 = chunk_in_slab(0)
            start_idx(kk0, 0, a)
            for k in range(K):
                b = k % 2
                wait_idx(b)
                if k + 1 < K:
                    start_idx(chunk_in_slab(k + 1), 1 - b, a)
                if not first or k >= 2:
                    wait_out(b)
                compute(b)
                start_out(si * K + chunk_in_slab(k), b)

            if prefetch:
                @pl.when(s == 0)
                def _():
                    slab_fetch_wait(1 - a)

            if not last:
                plsc.subcore_barrier()

        @pl.when(s == 0)
        def _():
            slab_fetch_start(0, 0)

        pltpu.sync_copy(table_hbm.at[f], tab_v)

        @pl.when(s == 0)
        def _():
            slab_fetch_wait(0)

        plsc.subcore_barrier()

        do_slab(jnp.int32(0), 0, True, True, False)

        @pl.loop(1, nslab - 2, step=2)
        def _(t):
            do_slab(t, 1, True, False, False)
            do_slab(t + 1, 0, True, False, False)

        do_slab(jnp.int32(nslab - 2), 1, True, False, False)
        do_slab(jnp.int32(nslab - 1), 0, False, False, True)

        for b in range(2):
            wait_out(b)

    out_t = gather_kernel(idx, table_t)
    return out_t.T
